# Initial kernel scaffold; baseline (speedup 1.0000x reference)
import jax
import jax.numpy as jnp
from jax.experimental import pallas as pl


def kernel(Loc, Cls, Seg, gt_box_batch, df_box_batch, idx_batch, cls_batch, bat_s, mining, seg_label):
    raise NotImplementedError("write your pallas kernel here")


# speedup vs baseline: 15.4457x; 15.4457x over previous
import jax
import jax.numpy as jnp
from jax import lax
from jax.experimental import pallas as pl
from jax.experimental.pallas import tpu as pltpu
from jax.experimental.pallas import tpu_sc as plsc

_MAP_SIZES = [64, 32, 16, 8, 4, 2]
_NUM_BOXES = [4, 6, 6, 6, 6, 4]
_OFF = 4
_LOC_VAR = 0.1

_CLSBASE = [0, 16384, 22528, 24064, 24448, 24544]
_TOTC = 24560
_TOTL = 4 * _TOTC

_NC, _NS = 2, 16
_B, _M = 32, 512


def _sc_build(idx2, clsb, gt2, df2, iv16):
    mesh = plsc.VectorSubcoreMesh(
        core_axis_name="c", subcore_axis_name="s",
        num_cores=_NC, num_subcores=_NS)

    @pl.kernel(
        out_type=(
            jax.ShapeDtypeStruct((_B, _TOTC), jnp.int32),
            jax.ShapeDtypeStruct((_B, _TOTL), jnp.float32),
        ),
        mesh=mesh,
        compiler_params=pltpu.CompilerParams(needs_layout_passes=False),
        scratch_types=[
            pltpu.VMEM((_TOTC,), jnp.int32),
            pltpu.VMEM((_TOTL,), jnp.float32),
            pltpu.VMEM((4 * _M,), jnp.int32),
            pltpu.VMEM((_M,), jnp.int32),
            pltpu.VMEM((4 * _M,), jnp.float32),
            pltpu.VMEM((4 * _M,), jnp.float32),
            pltpu.VMEM((16,), jnp.int32),
            pltpu.VMEM((64,), jnp.int32),
        ],
    )
    def build(idx_h, cls_h, gt_h, df_h, iv_h, outc_h, outl_h,
              outc_v, outl_v, idx_v, cls_v, gt_v, df_v, iv_v, tmp_v):
        b = lax.axis_index("s") * _NC + lax.axis_index("c")

        pltpu.sync_copy(idx_h.at[b], idx_v)
        pltpu.sync_copy(cls_h.at[b], cls_v)
        pltpu.sync_copy(gt_h.at[b], gt_v)
        pltpu.sync_copy(df_h.at[b], df_v)
        pltpu.sync_copy(iv_h, iv_v)

        ivv = iv_v[...]
        zf = jnp.zeros((16,), jnp.float32)

        def ms_c(i, carry):
            outc_v[pl.ds(i * 16, 16)] = ivv
            return carry
        lax.fori_loop(0, _TOTC // 16, ms_c, 0)

        def ms_l(i, carry):
            outl_v[pl.ds(i * 16, 16)] = zf
            return carry
        lax.fori_loop(0, _TOTL // 16, ms_l, 0)

        def lv(i, carry):
            g = gt_v[pl.ds(i * 16, 16)]
            d = df_v[pl.ds(i * 16, 16)]
            gt_v[pl.ds(i * 16, 16)] = (g - d) / jnp.float32(_LOC_VAR)
            return carry
        lax.fori_loop(0, (4 * _M) // 16, lv, 0)

        lane = lax.iota(jnp.int32, 16)
        six = jnp.full((16,), 6, jnp.int32)

        def group(g, carry):
            base = g * 16
            e4 = (base + lane) * 4
            l1 = plsc.load_gather(idx_v, [e4 + 1])
            pos = plsc.load_gather(idx_v, [e4 + 2])
            box = plsc.load_gather(idx_v, [e4 + 3])
            l = lax.rem(l1, six)
            s2 = jnp.right_shift(jnp.int32(4096), 2 * l)
            p = pos & (s2 - 1)
            nb = jnp.where((l == 0) | (l == 5), jnp.int32(4), jnp.int32(6))
            bx = lax.rem(box, nb)
            cb = jnp.where(l == 1, jnp.int32(_CLSBASE[1]), jnp.int32(0))
            cb = jnp.where(l == 2, jnp.int32(_CLSBASE[2]), cb)
            cb = jnp.where(l == 3, jnp.int32(_CLSBASE[3]), cb)
            cb = jnp.where(l == 4, jnp.int32(_CLSBASE[4]), cb)
            cb = jnp.where(l == 5, jnp.int32(_CLSBASE[5]), cb)
            cl = cb + bx * s2 + p
            la0 = 4 * (cl - p) + p

            skey, se = plsc.sort_key_val(cl * 16 + lane, lane)
            sa = jnp.right_shift(skey, 4)
            tmp_v[pl.ds(0, 16)] = sa
            tmp_v[pl.ds(16, 16)] = la0
            tmp_v[pl.ds(32, 16)] = s2
            nxt = plsc.load_gather(tmp_v, [jnp.minimum(lane + 1, 15)])
            keep = (sa != nxt) | (lane == 15)

            la0s = plsc.load_gather(tmp_v, [se + 16])
            shs = plsc.load_gather(tmp_v, [se + 32])
            cvals = plsc.load_gather(cls_v, [base + se])
            plsc.store_scatter(outc_v, [sa], cvals, mask=keep)
            se4 = (base + se) * 4
            for j in range(_OFF):
                lvj = plsc.load_gather(gt_v, [se4 + j])
                plsc.store_scatter(outl_v, [la0s + j * shs], lvj, mask=keep)
            return carry
        lax.fori_loop(0, _M // 16, group, 0)

        pltpu.sync_copy(outc_v, outc_h.at[b])
        pltpu.sync_copy(outl_v, outl_h.at[b])

    return build(idx2, clsb, gt2, df2, iv16)


def kernel(Loc, Cls, Seg, gt_box_batch, df_box_batch, idx_batch, cls_batch,
           bat_s, mining, seg_label):
    B, M = idx_batch.shape[0], idx_batch.shape[1]
    idx2 = idx_batch.reshape(B, 4 * M)
    gt2 = gt_box_batch.reshape(B, 4 * M)
    df2 = df_box_batch.reshape(B, 4 * M)
    iv = jnp.where(mining != 0, -1, 0).astype(jnp.int32)
    iv16 = jnp.broadcast_to(iv, (16,))

    outc, outl = _sc_build(idx2, cls_batch, gt2, df2, iv16)

    cls_out = []
    loc_out = []
    for l in range(6):
        S = _MAP_SIZES[l]
        NB = _NUM_BOXES[l]
        cb = _CLSBASE[l]
        cls_out.append(outc[:, cb:cb + NB * S * S].reshape(B, NB, S, S))
        loc_out.append(
            outl[:, 4 * cb:4 * cb + NB * _OFF * S * S].reshape(B, NB * _OFF, S, S))
    return tuple(cls_out) + tuple(loc_out)


# speedup vs baseline: 19.4435x; 1.2588x over previous
import jax
import jax.numpy as jnp
from jax import lax
from jax.experimental import pallas as pl
from jax.experimental.pallas import tpu as pltpu
from jax.experimental.pallas import tpu_sc as plsc

_MAP_SIZES = [64, 32, 16, 8, 4, 2]
_NUM_BOXES = [4, 6, 6, 6, 6, 4]
_OFF = 4
_LOC_VAR = 0.1

_CLSBASE = [0, 16384, 22528, 24064, 24448, 24544]
_CLSSZ = [16384, 6144, 1536, 384, 96, 16]
_TOTC = 24560
_TOTL = 4 * _TOTC

_NC, _NS = 2, 16
_B, _M = 32, 512


def _sc_build(idx2, clsb, gt2, df2, iv16):
    mesh = plsc.VectorSubcoreMesh(
        core_axis_name="c", subcore_axis_name="s",
        num_cores=_NC, num_subcores=_NS)

    out_type = tuple(
        jax.ShapeDtypeStruct((_B * _CLSSZ[l],), jnp.int32) for l in range(6)
    ) + tuple(
        jax.ShapeDtypeStruct((_B * 4 * _CLSSZ[l],), jnp.float32) for l in range(6)
    )

    @pl.kernel(
        out_type=out_type,
        mesh=mesh,
        compiler_params=pltpu.CompilerParams(needs_layout_passes=False),
        scratch_types=[
            pltpu.VMEM((_TOTC,), jnp.int32),
            pltpu.VMEM((_TOTL,), jnp.float32),
            pltpu.VMEM((4 * _M,), jnp.int32),
            pltpu.VMEM((_M,), jnp.int32),
            pltpu.VMEM((4 * _M,), jnp.float32),
            pltpu.VMEM((4 * _M,), jnp.float32),
            pltpu.VMEM((16,), jnp.int32),
            pltpu.SemaphoreType.DMA,
            pltpu.SemaphoreType.DMA,
        ],
    )
    def build(idx_h, cls_h, gt_h, df_h, iv_h,
              c0, c1, c2, c3, c4, c5, l0, l1_h, l2, l3, l4, l5,
              outc_v, outl_v, idx_v, cls_v, gt_v, df_v, iv_v,
              sem_in, sem_out):
        b = lax.axis_index("s") * _NC + lax.axis_index("c")

        stage = [
            pltpu.async_copy(idx_h.at[b], idx_v, sem_in),
            pltpu.async_copy(cls_h.at[b], cls_v, sem_in),
            pltpu.async_copy(gt_h.at[b], gt_v, sem_in),
            pltpu.async_copy(df_h.at[b], df_v, sem_in),
        ]
        pltpu.sync_copy(iv_h, iv_v)
        ivv = iv_v[...]
        zf = jnp.zeros((16,), jnp.float32)

        @plsc.parallel_loop(0, _TOTC, step=16, unroll=5)
        def _(i):
            outc_v[pl.ds(i, 16)] = ivv

        @plsc.parallel_loop(0, _TOTL, step=16, unroll=5)
        def _(i):
            outl_v[pl.ds(i, 16)] = zf

        for d in stage:
            d.wait()

        @plsc.parallel_loop(0, 4 * _M, step=16, unroll=4)
        def _(i):
            g = gt_v[pl.ds(i, 16)]
            d = df_v[pl.ds(i, 16)]
            gt_v[pl.ds(i, 16)] = (g - d) / jnp.float32(_LOC_VAR)

        lane = lax.iota(jnp.int32, 16)
        six = jnp.full((16,), 6, jnp.int32)
        nxt_i = jnp.minimum(lane + 1, 15)

        for g in range(_M // 16):
            base = g * 16
            e4 = (base + lane) * 4
            lyr = plsc.load_gather(idx_v, [e4 + 1])
            pos = plsc.load_gather(idx_v, [e4 + 2])
            box = plsc.load_gather(idx_v, [e4 + 3])
            l = lax.rem(lyr, six)
            s2 = jnp.right_shift(jnp.int32(4096), 2 * l)
            p = pos & (s2 - 1)
            nb = jnp.where((l == 0) | (l == 5), jnp.int32(4), jnp.int32(6))
            bx = lax.rem(box, nb)
            cb = jnp.where(l == 1, jnp.int32(_CLSBASE[1]), jnp.int32(0))
            cb = jnp.where(l == 2, jnp.int32(_CLSBASE[2]), cb)
            cb = jnp.where(l == 3, jnp.int32(_CLSBASE[3]), cb)
            cb = jnp.where(l == 4, jnp.int32(_CLSBASE[4]), cb)
            cb = jnp.where(l == 5, jnp.int32(_CLSBASE[5]), cb)
            cl = cb + bx * s2 + p
            la0 = 4 * (cl - p) + p

            skey, se = plsc.sort_key_val(cl * 16 + lane, lane)
            sa = jnp.right_shift(skey, 4)
            nxt = jnp.take_along_axis(sa, nxt_i, axis=0, mode="promise_in_bounds")
            keep = (sa != nxt) | (lane == 15)

            la0s = jnp.take_along_axis(la0, se, axis=0, mode="promise_in_bounds")
            shs = jnp.take_along_axis(s2, se, axis=0, mode="promise_in_bounds")
            cvals = plsc.load_gather(cls_v, [base + se])
            plsc.store_scatter(outc_v, [sa], cvals, mask=keep)
            se4 = (base + se) * 4
            for j in range(_OFF):
                lvj = plsc.load_gather(gt_v, [se4 + j])
                plsc.store_scatter(outl_v, [la0s + j * shs], lvj, mask=keep)

        outs_c = [c0, c1, c2, c3, c4, c5]
        outs_l = [l0, l1_h, l2, l3, l4, l5]
        wb = []
        for l in range(6):
            cbase, csz = _CLSBASE[l], _CLSSZ[l]
            wb.append(pltpu.async_copy(
                outc_v.at[pl.ds(cbase, csz)],
                outs_c[l].at[pl.ds(b * csz, csz)], sem_out))
            wb.append(pltpu.async_copy(
                outl_v.at[pl.ds(4 * cbase, 4 * csz)],
                outs_l[l].at[pl.ds(b * 4 * csz, 4 * csz)], sem_out))
        for d in wb:
            d.wait()

    return build(idx2, clsb, gt2, df2, iv16)


def kernel(Loc, Cls, Seg, gt_box_batch, df_box_batch, idx_batch, cls_batch,
           bat_s, mining, seg_label):
    B, M = idx_batch.shape[0], idx_batch.shape[1]
    idx2 = idx_batch.reshape(B, 4 * M)
    gt2 = gt_box_batch.reshape(B, 4 * M)
    df2 = df_box_batch.reshape(B, 4 * M)
    iv = jnp.where(mining != 0, -1, 0).astype(jnp.int32)
    iv16 = jnp.broadcast_to(iv, (16,))

    outs = _sc_build(idx2, cls_batch, gt2, df2, iv16)

    res = []
    for l in range(6):
        S, NB = _MAP_SIZES[l], _NUM_BOXES[l]
        res.append(outs[l].reshape(B, NB, S, S))
    for l in range(6):
        S, NB = _MAP_SIZES[l], _NUM_BOXES[l]
        res.append(outs[6 + l].reshape(B, NB * _OFF, S, S))
    return tuple(res)


# speedup vs baseline: 20.1267x; 1.0351x over previous
import jax
import jax.numpy as jnp
from jax import lax
from jax.experimental import pallas as pl
from jax.experimental.pallas import tpu as pltpu
from jax.experimental.pallas import tpu_sc as plsc

_MAP_SIZES = [64, 32, 16, 8, 4, 2]
_NUM_BOXES = [4, 6, 6, 6, 6, 4]
_OFF = 4
_LOC_VAR = 0.1

_CLSSZ = [16384, 6144, 1536, 384, 96, 16]
_CLSBASE = [0, 16384, 22528, 24064, 24448, 24544]
_TOTC = 24560
_TOTL = 4 * _TOTC

_NC, _NS = 2, 16
_B, _M = 32, 512


def _entry_addrs(idx_v, base, lane, six):
    e4 = (base + lane) * 4
    lyr = plsc.load_gather(idx_v, [e4 + 1])
    pos = plsc.load_gather(idx_v, [e4 + 2])
    box = plsc.load_gather(idx_v, [e4 + 3])
    l = lax.rem(lyr, six)
    s2 = jnp.right_shift(jnp.int32(4096), 2 * l)
    p = pos & (s2 - 1)
    nb = jnp.where((l == 0) | (l == 5), jnp.int32(4), jnp.int32(6))
    bx = lax.rem(box, nb)
    cb = jnp.where(l == 1, jnp.int32(_CLSBASE[1]), jnp.int32(0))
    cb = jnp.where(l == 2, jnp.int32(_CLSBASE[2]), cb)
    cb = jnp.where(l == 3, jnp.int32(_CLSBASE[3]), cb)
    cb = jnp.where(l == 4, jnp.int32(_CLSBASE[4]), cb)
    cb = jnp.where(l == 5, jnp.int32(_CLSBASE[5]), cb)
    cl = cb + bx * s2 + p
    la0 = 4 * (cl - p) + p
    return cl, la0, s2


def _dedup(cl, lane, nxt_i):
    skey, se = plsc.sort_key_val(cl * 16 + lane, lane)
    sa = jnp.right_shift(skey, 4)
    nxt = jnp.take_along_axis(sa, nxt_i, axis=0, mode="promise_in_bounds")
    keep = (sa != nxt) | (lane == 15)
    return sa, se, keep


def _sc_build_cls(idx2, clsb, iv16):
    mesh = plsc.VectorSubcoreMesh(
        core_axis_name="c", subcore_axis_name="s",
        num_cores=_NC, num_subcores=_NS)

    @pl.kernel(
        out_type=tuple(
            jax.ShapeDtypeStruct((_B * _CLSSZ[l],), jnp.int32)
            for l in range(6)),
        mesh=mesh,
        compiler_params=pltpu.CompilerParams(needs_layout_passes=False),
        scratch_types=[
            pltpu.VMEM((_TOTC,), jnp.int32),
            pltpu.VMEM((4 * _M,), jnp.int32),
            pltpu.VMEM((_M,), jnp.int32),
            pltpu.VMEM((16,), jnp.int32),
            pltpu.SemaphoreType.DMA,
            pltpu.SemaphoreType.DMA,
        ],
    )
    def build(idx_h, cls_h, iv_h, c0, c1, c2, c3, c4, c5,
              outc_v, idx_v, cls_v, iv_v, sem_in, sem_out):
        b = lax.axis_index("s") * _NC + lax.axis_index("c")

        stage = [
            pltpu.async_copy(idx_h.at[b], idx_v, sem_in),
            pltpu.async_copy(cls_h.at[b], cls_v, sem_in),
        ]
        pltpu.sync_copy(iv_h, iv_v)
        ivv = iv_v[...]

        @plsc.parallel_loop(0, _TOTC, step=16, unroll=5)
        def _(i):
            outc_v[pl.ds(i, 16)] = ivv

        for d in stage:
            d.wait()

        lane = lax.iota(jnp.int32, 16)
        six = jnp.full((16,), 6, jnp.int32)
        nxt_i = jnp.minimum(lane + 1, 15)

        for g in range(_M // 16):
            base = g * 16
            cl, _, _ = _entry_addrs(idx_v, base, lane, six)
            sa, se, keep = _dedup(cl, lane, nxt_i)
            cvals = plsc.load_gather(cls_v, [base + se])
            plsc.store_scatter(outc_v, [sa], cvals, mask=keep)

        outs = [c0, c1, c2, c3, c4, c5]
        wb = []
        for l in range(6):
            cbase, csz = _CLSBASE[l], _CLSSZ[l]
            wb.append(pltpu.async_copy(
                outc_v.at[pl.ds(cbase, csz)],
                outs[l].at[pl.ds(b * csz, csz)], sem_out))
        for d in wb:
            d.wait()

    return build(idx2, clsb, iv16)


def _sc_build_loc(idx2, gt2, df2):
    mesh = plsc.VectorSubcoreMesh(
        core_axis_name="c", subcore_axis_name="s",
        num_cores=_NC, num_subcores=_NS)

    @pl.kernel(
        out_type=tuple(
            jax.ShapeDtypeStruct((_B * 4 * _CLSSZ[l],), jnp.float32)
            for l in range(6)),
        mesh=mesh,
        compiler_params=pltpu.CompilerParams(needs_layout_passes=False),
        scratch_types=[
            pltpu.VMEM((_TOTL,), jnp.float32),
            pltpu.VMEM((4 * _M,), jnp.int32),
            pltpu.VMEM((4 * _M,), jnp.float32),
            pltpu.VMEM((4 * _M,), jnp.float32),
            pltpu.SemaphoreType.DMA,
            pltpu.SemaphoreType.DMA,
        ],
    )
    def build(idx_h, gt_h, df_h, l0, l1_h, l2, l3, l4, l5,
              outl_v, idx_v, gt_v, df_v, sem_in, sem_out):
        b = lax.axis_index("s") * _NC + lax.axis_index("c")

        stage = [
            pltpu.async_copy(idx_h.at[b], idx_v, sem_in),
            pltpu.async_copy(gt_h.at[b], gt_v, sem_in),
            pltpu.async_copy(df_h.at[b], df_v, sem_in),
        ]
        zf = jnp.zeros((16,), jnp.float32)

        @plsc.parallel_loop(0, _TOTL, step=16, unroll=5)
        def _(i):
            outl_v[pl.ds(i, 16)] = zf

        for d in stage:
            d.wait()

        @plsc.parallel_loop(0, 4 * _M, step=16, unroll=4)
        def _(i):
            g = gt_v[pl.ds(i, 16)]
            d = df_v[pl.ds(i, 16)]
            gt_v[pl.ds(i, 16)] = (g - d) / jnp.float32(_LOC_VAR)

        lane = lax.iota(jnp.int32, 16)
        six = jnp.full((16,), 6, jnp.int32)
        nxt_i = jnp.minimum(lane + 1, 15)

        for g in range(_M // 16):
            base = g * 16
            cl, la0, s2 = _entry_addrs(idx_v, base, lane, six)
            _, se, keep = _dedup(cl, lane, nxt_i)
            la0s = jnp.take_along_axis(la0, se, axis=0,
                                       mode="promise_in_bounds")
            shs = jnp.take_along_axis(s2, se, axis=0,
                                      mode="promise_in_bounds")
            se4 = (base + se) * 4
            for j in range(_OFF):
                lvj = plsc.load_gather(gt_v, [se4 + j])
                plsc.store_scatter(outl_v, [la0s + j * shs], lvj, mask=keep)

        outs = [l0, l1_h, l2, l3, l4, l5]
        wb = []
        for l in range(6):
            lbase, lsz = 4 * _CLSBASE[l], 4 * _CLSSZ[l]
            wb.append(pltpu.async_copy(
                outl_v.at[pl.ds(lbase, lsz)],
                outs[l].at[pl.ds(b * lsz, lsz)], sem_out))
        for d in wb:
            d.wait()

    return build(idx2, gt2, df2)


def kernel(Loc, Cls, Seg, gt_box_batch, df_box_batch, idx_batch, cls_batch,
           bat_s, mining, seg_label):
    B, M = idx_batch.shape[0], idx_batch.shape[1]
    idx2 = idx_batch.reshape(B, 4 * M)
    gt2 = gt_box_batch.reshape(B, 4 * M)
    df2 = df_box_batch.reshape(B, 4 * M)
    iv = jnp.where(mining != 0, -1, 0).astype(jnp.int32)
    iv16 = jnp.broadcast_to(iv, (16,))

    outs_c = _sc_build_cls(idx2, cls_batch, iv16)
    outs_l = _sc_build_loc(idx2, gt2, df2)

    res = []
    for l in range(6):
        S, NB = _MAP_SIZES[l], _NUM_BOXES[l]
        res.append(outs_c[l].reshape(B, NB, S, S))
    for l in range(6):
        S, NB = _MAP_SIZES[l], _NUM_BOXES[l]
        res.append(outs_l[l].reshape(B, NB * _OFF, S, S))
    return tuple(res)
